# Initial kernel scaffold; baseline (speedup 1.0000x reference)
import jax
import jax.numpy as jnp
from jax.experimental import pallas as pl


def kernel(text_batch, emb_table):
    raise NotImplementedError("write your pallas kernel here")


# speedup vs baseline: 12.9154x; 12.9154x over previous
import jax
import jax.numpy as jnp
from jax.experimental import pallas as pl

TOKEN_LENGTH = 32
HIDDEN_SIZE = 128
BLOCK_ROWS = 128


def _body(x_ref, tab_ref, emb_ref, logit_ref):
    amp = x_ref[...].astype(jnp.float32) * 2.0 - 1.0
    tab = tab_ref[...]
    emb_ref[...] = amp[:, :, None] * tab[None, :, :]
    rowsum = jnp.sum(tab, axis=1)
    logit_ref[...] = amp * rowsum[None, :]


def kernel(text_batch, emb_table):
    B, S, L = text_batch.shape
    H = emb_table.shape[1]
    N = B * S
    x = text_batch.reshape(N, L)
    R = BLOCK_ROWS
    grid = (N // R,)
    emb_flat, logit_flat = pl.pallas_call(
        _body,
        grid=grid,
        in_specs=[
            pl.BlockSpec((R, L), lambda i: (i, 0)),
            pl.BlockSpec((L, H), lambda i: (0, 0)),
        ],
        out_specs=[
            pl.BlockSpec((R, L, H), lambda i: (i, 0, 0)),
            pl.BlockSpec((R, L), lambda i: (i, 0)),
        ],
        out_shape=[
            jax.ShapeDtypeStruct((N, L, H), jnp.float32),
            jax.ShapeDtypeStruct((N, L), jnp.float32),
        ],
    )(x, emb_table)
    embeddings = emb_flat.reshape(B, S, L, H)
    logit_prime = logit_flat.reshape(B, S, L, 1)
    return embeddings, logit_prime


# speedup vs baseline: 18.1965x; 1.4089x over previous
import jax
import jax.numpy as jnp
from jax.experimental import pallas as pl

TOKEN_LENGTH = 32
HIDDEN_SIZE = 128
BLOCK_ROWS = 512


def _body(x_ref, tab_ref, emb_ref, logit_ref):
    amp = x_ref[...].astype(jnp.float32) * 2.0 - 1.0
    tab = tab_ref[...]
    emb_ref[...] = amp[:, :, None] * tab[None, :, :]
    rowsum = jnp.sum(tab, axis=1)
    logit_ref[...] = amp * rowsum[None, :]


def kernel(text_batch, emb_table):
    B, S, L = text_batch.shape
    H = emb_table.shape[1]
    N = B * S
    x = text_batch.reshape(N, L)
    R = BLOCK_ROWS
    grid = (N // R,)
    emb_flat, logit_flat = pl.pallas_call(
        _body,
        grid=grid,
        in_specs=[
            pl.BlockSpec((R, L), lambda i: (i, 0)),
            pl.BlockSpec((L, H), lambda i: (0, 0)),
        ],
        out_specs=[
            pl.BlockSpec((R, L, H), lambda i: (i, 0, 0)),
            pl.BlockSpec((R, L), lambda i: (i, 0)),
        ],
        out_shape=[
            jax.ShapeDtypeStruct((N, L, H), jnp.float32),
            jax.ShapeDtypeStruct((N, L), jnp.float32),
        ],
    )(x, emb_table)
    embeddings = emb_flat.reshape(B, S, L, H)
    logit_prime = logit_flat.reshape(B, S, L, 1)
    return embeddings, logit_prime
